# baseline (speedup 1.0000x reference)
import functools

import jax
import jax.numpy as jnp
from jax import lax
from jax.experimental import pallas as pl
from jax.experimental.pallas import tpu as pltpu
from jax.experimental.pallas import tpu_sc as plsc

EMB = 64
BATCH = 16384

NC = 2
NS = 16
NW = NC * NS
B_PER_W = BATCH // NW
CHUNK = 128


def _sc_gather1(idxs, tab):
    mesh = plsc.VectorSubcoreMesh(core_axis_name="c", subcore_axis_name="s")
    row_t = jax.ShapeDtypeStruct((BATCH, 2 * EMB), jnp.float32)

    @functools.partial(
        pl.kernel,
        mesh=mesh,
        out_type=row_t,
        scratch_types=[
            pltpu.VMEM((B_PER_W,), jnp.int32),
            pltpu.VMEM((CHUNK, 2 * EMB), jnp.float32),
            pltpu.VMEM((CHUNK, 2 * EMB), jnp.float32),
            pltpu.SemaphoreType.DMA,
            pltpu.SemaphoreType.DMA,
        ],
    )
    def k(idx_hbm, t_hbm, o_hbm, idx, bufa, bufb, gs0, gs1):
        wid = lax.axis_index("s") * NC + lax.axis_index("c")
        base = wid * B_PER_W
        sl = pl.ds(base, B_PER_W)
        pltpu.sync_copy(idx_hbm.at[sl], idx)

        nch = B_PER_W // CHUNK
        bufs = (bufa, bufb)
        gsems = (gs0, gs1)
        slots = [None, None]
        for n in (0, 1):
            slots[n] = pltpu.async_copy(
                t_hbm.at[idx.at[pl.ds(n * CHUNK, CHUNK)]], bufs[n], gsems[n])
        for n in range(nch):
            b = n % 2
            off = n * CHUNK
            slots[b].wait()
            pltpu.sync_copy(bufs[b], o_hbm.at[pl.ds(base + off, CHUNK)])
            if n + 2 < nch:
                noff = (n + 2) * CHUNK
                slots[b] = pltpu.async_copy(
                    t_hbm.at[idx.at[pl.ds(noff, CHUNK)]], bufs[b], gsems[b])

    return k(idxs, tab)


N_ROWS = 100000
TR = 10000


def _concat_body(a_ref, b_ref, u_ref):
    u_ref[...] = jnp.concatenate([a_ref[...], b_ref[...]], axis=1)


def _tc_concat1(a, b):
    half_spec = pl.BlockSpec((TR, EMB), lambda i: (i, 0))
    out_spec = pl.BlockSpec((TR, 2 * EMB), lambda i: (i, 0))
    out_t = jax.ShapeDtypeStruct((N_ROWS, 2 * EMB), jnp.float32)
    return pl.pallas_call(
        _concat_body,
        grid=(N_ROWS // TR,),
        in_specs=[half_spec, half_spec],
        out_specs=out_spec,
        out_shape=out_t,
    )(a, b)


BR = 4096


def _tc_dense_body(u_ref, i_ref,
                   w1a_ref, w1b_ref, b1_ref, w2_ref, b2_ref, w3_ref, b3_ref,
                   wmg_ref, wmh_ref, bm_ref, out_ref):
    f32 = jnp.float32
    ug = u_ref[:, :EMB]
    um = u_ref[:, EMB:]
    ig = i_ref[:, :EMB]
    im = i_ref[:, EMB:]

    h1 = jnp.dot(um, w1a_ref[...], preferred_element_type=f32)
    h1 += jnp.dot(im, w1b_ref[...], preferred_element_type=f32)
    h1 = jnp.maximum(h1 + b1_ref[...], 0.0)
    h2 = jnp.maximum(
        jnp.dot(h1, w2_ref[...], preferred_element_type=f32) + b2_ref[...], 0.0)
    h3 = jnp.maximum(
        jnp.dot(h2, w3_ref[...], preferred_element_type=f32) + b3_ref[...], 0.0)
    g = ug * ig
    r = jnp.dot(g, wmg_ref[...], preferred_element_type=f32)
    r += jnp.dot(h3, wmh_ref[...], preferred_element_type=f32)
    out_ref[...] = r + bm_ref[...]


def _tc_dense(u_rows, i_rows, W1, b1, W2, b2, W3, b3, Wm, bm):
    w1a = W1[:, :EMB].T
    w1b = W1[:, EMB:].T
    w2 = W2.T
    w3 = W3.T
    wmg = Wm[:, :EMB].T
    wmh = Wm[:, EMB:].T
    b1r = b1.reshape(1, -1)
    b2r = b2.reshape(1, -1)
    b3r = b3.reshape(1, -1)
    bmr = bm.reshape(1, 1)

    row_spec = pl.BlockSpec((BR, 2 * EMB), lambda i: (i, 0))
    full = lambda a: pl.BlockSpec(a.shape, lambda i: (0,) * a.ndim)

    out = pl.pallas_call(
        _tc_dense_body,
        grid=(BATCH // BR,),
        in_specs=[row_spec, row_spec,
                  full(w1a), full(w1b), full(b1r), full(w2), full(b2r),
                  full(w3), full(b3r), full(wmg), full(wmh), full(bmr)],
        out_specs=pl.BlockSpec((BR, 1), lambda i: (i, 0)),
        out_shape=jax.ShapeDtypeStruct((BATCH, 1), jnp.float32),
    )(u_rows, i_rows, w1a, w1b, b1r, w2, b2r, w3, b3r, wmg, wmh, bmr)
    return jnp.squeeze(out, axis=-1)


def kernel(users, items, user_GMF, item_GMF, user_MLP, item_MLP,
           W1, b1, W2, b2, W3, b3, Wm, bm):
    u_tab = _tc_concat1(user_GMF, user_MLP)
    u_rows = _sc_gather1(users, u_tab)
    i_tab = _tc_concat1(item_GMF, item_MLP)
    i_rows = _sc_gather1(items, i_tab)
    return _tc_dense(u_rows, i_rows, W1, b1, W2, b2, W3, b3, Wm, bm)


# speedup vs baseline: 1.2514x; 1.2514x over previous
import functools

import jax
import jax.numpy as jnp
from jax import lax
from jax.experimental import pallas as pl
from jax.experimental.pallas import tpu as pltpu
from jax.experimental.pallas import tpu_sc as plsc

EMB = 64
BATCH = 16384

NC = 2
NS = 16
NW = NC * NS
B_PER_W = BATCH // NW
CHUNK = 128


def _sc_gather1(idxs, tab):
    mesh = plsc.VectorSubcoreMesh(core_axis_name="c", subcore_axis_name="s")
    row_t = jax.ShapeDtypeStruct((BATCH, 2 * EMB), jnp.float32)

    @functools.partial(
        pl.kernel,
        mesh=mesh,
        out_type=row_t,
        scratch_types=[
            pltpu.VMEM((B_PER_W,), jnp.int32),
            pltpu.VMEM((CHUNK, 2 * EMB), jnp.float32),
            pltpu.VMEM((CHUNK, 2 * EMB), jnp.float32),
            pltpu.SemaphoreType.DMA,
            pltpu.SemaphoreType.DMA,
        ],
    )
    def k(idx_hbm, t_hbm, o_hbm, idx, bufa, bufb, gs0, gs1):
        wid = lax.axis_index("s") * NC + lax.axis_index("c")
        base = wid * B_PER_W
        sl = pl.ds(base, B_PER_W)
        pltpu.sync_copy(idx_hbm.at[sl], idx)

        nch = B_PER_W // CHUNK
        bufs = (bufa, bufb)
        gsems = (gs0, gs1)
        slots = [None, None]
        for n in (0, 1):
            slots[n] = pltpu.async_copy(
                t_hbm.at[idx.at[pl.ds(n * CHUNK, CHUNK)]], bufs[n], gsems[n])
        for n in range(nch):
            b = n % 2
            off = n * CHUNK
            slots[b].wait()
            pltpu.sync_copy(bufs[b], o_hbm.at[pl.ds(base + off, CHUNK)])
            if n + 2 < nch:
                noff = (n + 2) * CHUNK
                slots[b] = pltpu.async_copy(
                    t_hbm.at[idx.at[pl.ds(noff, CHUNK)]], bufs[b], gsems[b])

    return k(idxs, tab)


N_ROWS = 100000
TR = 10000


def _concat_body(a_ref, b_ref, u_ref):
    u_ref[...] = jnp.concatenate([a_ref[...], b_ref[...]], axis=1)


def _tc_concat1(a, b):
    half_spec = pl.BlockSpec((TR, EMB), lambda i: (i, 0))
    out_spec = pl.BlockSpec((TR, 2 * EMB), lambda i: (i, 0))
    out_t = jax.ShapeDtypeStruct((N_ROWS, 2 * EMB), jnp.float32)
    return pl.pallas_call(
        _concat_body,
        grid=(N_ROWS // TR,),
        in_specs=[half_spec, half_spec],
        out_specs=out_spec,
        out_shape=out_t,
    )(a, b)


BR = 4096


def _tc_dense_body(u_ref, i_ref,
                   w1a_ref, w1b_ref, b1_ref, w2_ref, b2_ref, w3_ref, b3_ref,
                   wmg_ref, wmh_ref, bm_ref, out_ref):
    f32 = jnp.float32
    ug = u_ref[:, :EMB]
    um = u_ref[:, EMB:]
    ig = i_ref[:, :EMB]
    im = i_ref[:, EMB:]

    h1 = jnp.dot(um, w1a_ref[...], preferred_element_type=f32)
    h1 += jnp.dot(im, w1b_ref[...], preferred_element_type=f32)
    h1 = jnp.maximum(h1 + b1_ref[...], 0.0)
    h2 = jnp.maximum(
        jnp.dot(h1, w2_ref[...], preferred_element_type=f32) + b2_ref[...], 0.0)
    h3 = jnp.maximum(
        jnp.dot(h2, w3_ref[...], preferred_element_type=f32) + b3_ref[...], 0.0)
    g = ug * ig
    r = jnp.dot(g, wmg_ref[...], preferred_element_type=f32)
    r += jnp.dot(h3, wmh_ref[...], preferred_element_type=f32)
    out_ref[...] = r + bm_ref[...]


def _tc_dense(u_rows, i_rows, W1, b1, W2, b2, W3, b3, Wm, bm):
    w1a = W1[:, :EMB].T
    w1b = W1[:, EMB:].T
    w2 = W2.T
    w3 = W3.T
    wmg = Wm[:, :EMB].T
    wmh = Wm[:, EMB:].T
    b1r = b1.reshape(1, -1)
    b2r = b2.reshape(1, -1)
    b3r = b3.reshape(1, -1)
    bmr = bm.reshape(1, 1)

    row_spec = pl.BlockSpec((BR, 2 * EMB), lambda i: (i, 0))
    full = lambda a: pl.BlockSpec(a.shape, lambda i: (0,) * a.ndim)

    out = pl.pallas_call(
        _tc_dense_body,
        grid=(BATCH // BR,),
        in_specs=[row_spec, row_spec,
                  full(w1a), full(w1b), full(b1r), full(w2), full(b2r),
                  full(w3), full(b3r), full(wmg), full(wmh), full(bmr)],
        out_specs=pl.BlockSpec((BR, 1), lambda i: (i, 0)),
        out_shape=jax.ShapeDtypeStruct((BATCH, 1), jnp.float32),
    )(u_rows, i_rows, w1a, w1b, b1r, w2, b2r, w3, b3r, wmg, wmh, bmr)
    return jnp.squeeze(out, axis=-1)


def kernel(users, items, user_GMF, item_GMF, user_MLP, item_MLP,
           W1, b1, W2, b2, W3, b3, Wm, bm):
    u_tab = jnp.concatenate([user_GMF, user_MLP], axis=1)
    i_tab = jnp.concatenate([item_GMF, item_MLP], axis=1)
    u_rows = _sc_gather1(users, u_tab)
    i_rows = _sc_gather1(items, i_tab)
    return _tc_dense(u_rows, i_rows, W1, b1, W2, b2, W3, b3, Wm, bm)


# speedup vs baseline: 1.2604x; 1.0071x over previous
import functools

import jax
import jax.numpy as jnp
from jax import lax
from jax.experimental import pallas as pl
from jax.experimental.pallas import tpu as pltpu
from jax.experimental.pallas import tpu_sc as plsc

EMB = 64
BATCH = 16384

NC = 2
NS = 16
NW = NC * NS
B_PER_W = BATCH // NW
CHUNK = 256


def _sc_gather1(idxs, tab):
    mesh = plsc.VectorSubcoreMesh(core_axis_name="c", subcore_axis_name="s")
    row_t = jax.ShapeDtypeStruct((BATCH, 2 * EMB), jnp.float32)

    @functools.partial(
        pl.kernel,
        mesh=mesh,
        out_type=row_t,
        scratch_types=[
            pltpu.VMEM((B_PER_W,), jnp.int32),
            pltpu.VMEM((CHUNK, 2 * EMB), jnp.float32),
            pltpu.VMEM((CHUNK, 2 * EMB), jnp.float32),
            pltpu.SemaphoreType.DMA,
            pltpu.SemaphoreType.DMA,
        ],
    )
    def k(idx_hbm, t_hbm, o_hbm, idx, bufa, bufb, gs0, gs1):
        wid = lax.axis_index("s") * NC + lax.axis_index("c")
        base = wid * B_PER_W
        sl = pl.ds(base, B_PER_W)
        pltpu.sync_copy(idx_hbm.at[sl], idx)

        nch = B_PER_W // CHUNK
        bufs = (bufa, bufb)
        gsems = (gs0, gs1)
        slots = [None, None]
        for n in (0, 1):
            slots[n] = pltpu.async_copy(
                t_hbm.at[idx.at[pl.ds(n * CHUNK, CHUNK)]], bufs[n], gsems[n])
        for n in range(nch):
            b = n % 2
            off = n * CHUNK
            slots[b].wait()
            pltpu.sync_copy(bufs[b], o_hbm.at[pl.ds(base + off, CHUNK)])
            if n + 2 < nch:
                noff = (n + 2) * CHUNK
                slots[b] = pltpu.async_copy(
                    t_hbm.at[idx.at[pl.ds(noff, CHUNK)]], bufs[b], gsems[b])

    return k(idxs, tab)


N_ROWS = 100000
TR = 10000


def _concat_body(a_ref, b_ref, u_ref):
    u_ref[...] = jnp.concatenate([a_ref[...], b_ref[...]], axis=1)


def _tc_concat1(a, b):
    half_spec = pl.BlockSpec((TR, EMB), lambda i: (i, 0))
    out_spec = pl.BlockSpec((TR, 2 * EMB), lambda i: (i, 0))
    out_t = jax.ShapeDtypeStruct((N_ROWS, 2 * EMB), jnp.float32)
    return pl.pallas_call(
        _concat_body,
        grid=(N_ROWS // TR,),
        in_specs=[half_spec, half_spec],
        out_specs=out_spec,
        out_shape=out_t,
    )(a, b)


BR = 4096


def _tc_dense_body(u_ref, i_ref,
                   w1a_ref, w1b_ref, b1_ref, w2_ref, b2_ref, w3_ref, b3_ref,
                   wmg_ref, wmh_ref, bm_ref, out_ref):
    f32 = jnp.float32
    ug = u_ref[:, :EMB]
    um = u_ref[:, EMB:]
    ig = i_ref[:, :EMB]
    im = i_ref[:, EMB:]

    h1 = jnp.dot(um, w1a_ref[...], preferred_element_type=f32)
    h1 += jnp.dot(im, w1b_ref[...], preferred_element_type=f32)
    h1 = jnp.maximum(h1 + b1_ref[...], 0.0)
    h2 = jnp.maximum(
        jnp.dot(h1, w2_ref[...], preferred_element_type=f32) + b2_ref[...], 0.0)
    h3 = jnp.maximum(
        jnp.dot(h2, w3_ref[...], preferred_element_type=f32) + b3_ref[...], 0.0)
    g = ug * ig
    r = jnp.dot(g, wmg_ref[...], preferred_element_type=f32)
    r += jnp.dot(h3, wmh_ref[...], preferred_element_type=f32)
    out_ref[...] = r + bm_ref[...]


def _tc_dense(u_rows, i_rows, W1, b1, W2, b2, W3, b3, Wm, bm):
    w1a = W1[:, :EMB].T
    w1b = W1[:, EMB:].T
    w2 = W2.T
    w3 = W3.T
    wmg = Wm[:, :EMB].T
    wmh = Wm[:, EMB:].T
    b1r = b1.reshape(1, -1)
    b2r = b2.reshape(1, -1)
    b3r = b3.reshape(1, -1)
    bmr = bm.reshape(1, 1)

    row_spec = pl.BlockSpec((BR, 2 * EMB), lambda i: (i, 0))
    full = lambda a: pl.BlockSpec(a.shape, lambda i: (0,) * a.ndim)

    out = pl.pallas_call(
        _tc_dense_body,
        grid=(BATCH // BR,),
        in_specs=[row_spec, row_spec,
                  full(w1a), full(w1b), full(b1r), full(w2), full(b2r),
                  full(w3), full(b3r), full(wmg), full(wmh), full(bmr)],
        out_specs=pl.BlockSpec((BR, 1), lambda i: (i, 0)),
        out_shape=jax.ShapeDtypeStruct((BATCH, 1), jnp.float32),
    )(u_rows, i_rows, w1a, w1b, b1r, w2, b2r, w3, b3r, wmg, wmh, bmr)
    return jnp.squeeze(out, axis=-1)


def kernel(users, items, user_GMF, item_GMF, user_MLP, item_MLP,
           W1, b1, W2, b2, W3, b3, Wm, bm):
    u_tab = jnp.concatenate([user_GMF, user_MLP], axis=1)
    i_tab = jnp.concatenate([item_GMF, item_MLP], axis=1)
    u_rows = _sc_gather1(users, u_tab)
    i_rows = _sc_gather1(items, i_tab)
    return _tc_dense(u_rows, i_rows, W1, b1, W2, b2, W3, b3, Wm, bm)
